# baseline (speedup 1.0000x reference)
import functools

import jax
import jax.numpy as jnp
from jax import lax
from jax.experimental import pallas as pl
from jax.experimental.pallas import tpu as pltpu
from jax.experimental.pallas import tpu_sc as plsc

B, K, D, V = 64, 64, 4096, 8192


_F_DB = 128
_F_NB = D // _F_DB


def _mega_body(keys_ref, q_ref, wb1_ref, wb2_ref, dist_ref, bb_ref, w1a_ref,
               w1b_ref, b1_ref, w2_ref, b2_ref,
               w_ref, lam_ref, kd_acc, kcache, hq_scr, h_acc, wsr):
    i = pl.program_id(0)

    @pl.when(i < _F_NB)
    def _():
        x = keys_ref[...]
        @pl.when(i < _F_NB - 1)
        def _():
            kcache[jnp.minimum(i, _F_NB - 2)] = x.astype(jnp.bfloat16)
        s1 = jnp.sum(x, axis=1)
        part = jnp.sum(s1 * wb2_ref[...], axis=1, keepdims=True)

        @pl.when(i == 0)
        def _():
            kd_acc[...] = part

        @pl.when(i > 0)
        def _():
            kd_acc[...] += part

        hq_scr[i] = jnp.dot(q_ref[...].astype(jnp.bfloat16),
                            w1a_ref[...].astype(jnp.bfloat16),
                            preferred_element_type=jnp.float32
                            ).astype(jnp.bfloat16)

    @pl.when(i == _F_NB)
    def _():
        qd = jnp.sum(q_ref[...] * wb1_ref[...], axis=1,
                     keepdims=True)
        t2 = kd_acc[...] * (1.0 / K)
        bw = jnp.exp(qd + t2 + bb_ref[...])
        sd = -jnp.sqrt(dist_ref[...]) / bw
        m = jnp.max(sd, axis=1, keepdims=True)
        e = jnp.exp(sd - m)
        w_ref[...] = e / jnp.sum(e, axis=1, keepdims=True)

    @pl.when(i >= _F_NB)
    def _():
        r = i - _F_NB
        wloc = w_ref[...][:, :, None]

        @pl.when(r < _F_NB - 1)
        def _():
            xk = kcache[jnp.minimum(r, _F_NB - 2)].astype(jnp.float32)
            wsr[...] = jnp.sum(wloc * xk, axis=1)

        @pl.when(r == _F_NB - 1)
        def _():
            wsr[...] = jnp.sum(wloc * keys_ref[...], axis=1)

        part = jnp.dot(wsr[...].astype(jnp.bfloat16),
                       w1b_ref[...].astype(jnp.bfloat16),
                       preferred_element_type=jnp.float32)

        @pl.when(r == 0)
        def _():
            h_acc[...] = part

        @pl.when(r > 0)
        def _():
            h_acc[...] += part

        @pl.when(r == _F_NB - 1)
        def _():
            acc = None
            for j in range(_F_NB):
                sl = slice(j * _F_DB, (j + 1) * _F_DB)
                hj = jnp.maximum(hq_scr[j].astype(jnp.float32)
                                 + h_acc[:, sl] + b1_ref[:, sl], 0.0)
                pj = jnp.sum(hj * w2_ref[:, sl], axis=1, keepdims=True)
                acc = pj if acc is None else acc + pj
            lam_ref[...] = jax.nn.sigmoid(acc + b2_ref[...])


def _mega(keys, query, wb1, wb2_row, distances, bb11, W1, b1_row, W2, b2_11):
    return pl.pallas_call(
        _mega_body,
        grid=(2 * _F_NB,),
        in_specs=[
            pl.BlockSpec((B, K, _F_DB),
                         lambda i: (0, 0, jnp.minimum(i, _F_NB - 1))),
            pl.BlockSpec((B, D), lambda i: (0, 0)),
            pl.BlockSpec((1, D), lambda i: (0, 0)),
            pl.BlockSpec((1, _F_DB), lambda i: (0, jnp.minimum(i, _F_NB - 1))),
            pl.BlockSpec((B, K), lambda i: (0, 0)),
            pl.BlockSpec((1, 1), lambda i: (0, 0)),
            pl.BlockSpec((D, _F_DB),
                         lambda i: (0, jnp.minimum(i, _F_NB - 1))),
            pl.BlockSpec((_F_DB, D),
                         lambda i: (jnp.clip(i - _F_NB, 0, _F_NB - 1) + _F_NB,
                                    0)),
            pl.BlockSpec((1, D), lambda i: (0, 0)),
            pl.BlockSpec((1, D), lambda i: (0, 0)),
            pl.BlockSpec((1, 1), lambda i: (0, 0)),
        ],
        out_specs=[
            pl.BlockSpec((B, K), lambda i: (0, 0)),
            pl.BlockSpec((B, 1), lambda i: (0, 0)),
        ],
        out_shape=[
            jax.ShapeDtypeStruct((B, K), jnp.float32),
            jax.ShapeDtypeStruct((B, 1), jnp.float32),
        ],
        scratch_shapes=[
            pltpu.VMEM((B, 1), jnp.float32),
            pltpu.VMEM((_F_NB - 1, B, K, _F_DB), jnp.bfloat16),
            pltpu.VMEM((_F_NB, B, _F_DB), jnp.bfloat16),
            pltpu.VMEM((B, D), jnp.float32),
            pltpu.VMEM((B, _F_DB), jnp.float32),
        ],
    )(keys, query, wb1, wb2_row, distances, bb11, W1, W1, b1_row, W2.reshape(1, D), b2_11)


_NC, _NS = 2, 16
_RG = 16
_NVS = 8
_VSEG = V // _NVS
_NRG = B // _RG


@functools.partial(
    pl.kernel,
    mesh=plsc.VectorSubcoreMesh(core_axis_name="c", subcore_axis_name="s"),
    compiler_params=pltpu.CompilerParams(needs_layout_passes=False),
    out_type=jax.ShapeDtypeStruct((B, V), jnp.float32),
    scratch_types=[
        pltpu.VMEM((_RG, K), jnp.int32),
        pltpu.VMEM((_RG, K), jnp.float32),
        pltpu.VMEM((_RG, _VSEG), jnp.float32),
        pltpu.SemaphoreType.DMA,
        pltpu.SemaphoreType.DMA,
    ],
)
def _scatter(vals_hbm, w_hbm, out_hbm, idx_v, w_v, acc_v, sem1, sem2):
    wid = lax.axis_index("s") * _NC + lax.axis_index("c")
    rg = wid % _NRG
    vs = wid // _NRG
    r0 = rg * _RG
    lo = vs * _VSEG

    cp1 = pltpu.async_copy(vals_hbm.at[pl.ds(r0, _RG), :], idx_v, sem1)
    cp2 = pltpu.async_copy(w_hbm.at[pl.ds(r0, _RG), :], w_v, sem2)

    zeros16 = jnp.zeros((16,), jnp.float32)

    @pl.loop(0, _RG)
    def _(r):
        @pl.loop(0, _VSEG, step=16, unroll=8)
        def _(cc):
            acc_v[r, pl.ds(cc, 16)] = zeros16

    cp1.wait()
    cp2.wait()

    lane = lax.iota(jnp.int32, 16)

    @pl.loop(0, K, unroll=4)
    def _(k):
        kk = jnp.full((16,), k, jnp.int32)
        iv = plsc.load_gather(idx_v, [lane, kk])
        wv = plsc.load_gather(w_v, [lane, kk])
        local = iv - lo
        mask = (local >= 0) & (local < _VSEG)
        clamped = jnp.clip(local, 0, _VSEG - 1)
        plsc.addupdate_scatter(acc_v, [lane, clamped], wv, mask=mask)

    pltpu.sync_copy(acc_v, out_hbm.at[pl.ds(r0, _RG), pl.ds(lo, _VSEG)])


def kernel(query, keys, distances, values, Wb, bb, W1, b1, W2, b2):
    wb1 = Wb[:D].reshape(1, D)
    wb2_row = Wb[D:].reshape(1, D)
    bb11 = bb.reshape(1, 1)
    b1_row = b1.reshape(1, D)
    b2_11 = b2.reshape(1, 1)

    w, lam = _mega(keys, query, wb1, wb2_row, distances, bb11, W1,
                   b1_row, W2, b2_11)

    vals2d = values[..., 0].astype(jnp.int32)
    probs = _scatter(vals2d, w)
    return probs, lam


# speedup vs baseline: 1.2129x; 1.2129x over previous
import functools

import jax
import jax.numpy as jnp
from jax import lax
from jax.experimental import pallas as pl
from jax.experimental.pallas import tpu as pltpu
from jax.experimental.pallas import tpu_sc as plsc

B, K, D, V = 64, 64, 4096, 8192


_F_DB = 256
_F_NB = D // _F_DB


def _mega_body(keys_ref, q_ref, wb1_ref, wb2_ref, dist_ref, bb_ref, w1a_ref,
               w1b_ref, b1_ref, w2_ref, b2_ref,
               w_ref, lam_ref, kd_acc, kcache, hh, wsr):
    i = pl.program_id(0)

    @pl.when(i < _F_NB)
    def _():
        x = keys_ref[...]

        @pl.when(i < _F_NB - 2)
        def _():
            kcache[jnp.minimum(i, _F_NB - 3)] = x.astype(jnp.bfloat16)

        qp = q_ref[...]
        s1 = jnp.sum(x, axis=1)
        part = (jnp.sum(s1 * wb2_ref[...], axis=1, keepdims=True) * (1.0 / K)
                + jnp.sum(qp * wb1_ref[...], axis=1, keepdims=True))

        @pl.when(i == 0)
        def _():
            kd_acc[...] = part

        @pl.when(i > 0)
        def _():
            kd_acc[...] += part

        pm = jnp.dot(qp.astype(jnp.bfloat16),
                     w1a_ref[...].astype(jnp.bfloat16),
                     preferred_element_type=jnp.float32)

        @pl.when(i == 0)
        def _():
            hh[...] = pm

        @pl.when(i > 0)
        def _():
            hh[...] += pm

    @pl.when(i == _F_NB)
    def _():
        bw = jnp.exp(kd_acc[...] + bb_ref[...])
        sd = -jnp.sqrt(dist_ref[...]) / bw
        m = jnp.max(sd, axis=1, keepdims=True)
        e = jnp.exp(sd - m)
        w_ref[...] = e / jnp.sum(e, axis=1, keepdims=True)

    @pl.when(i >= _F_NB)
    def _():
        r = i - _F_NB

        @pl.when(r <= 1)
        def _():
            wloc = w_ref[...][:, :, None]
            wsr[...] = jnp.sum(wloc * keys_ref[...], axis=1)

        @pl.when(r > 1)
        def _():
            wloc = w_ref[...][:, :, None]
            xk = kcache[jnp.clip(_F_NB - 1 - r, 0, _F_NB - 3)]
            wsr[...] = jnp.sum(wloc * xk.astype(jnp.float32), axis=1)

        hh[...] += jnp.dot(wsr[...].astype(jnp.bfloat16),
                           w1b_ref[...].astype(jnp.bfloat16),
                           preferred_element_type=jnp.float32)

        @pl.when(r == _F_NB - 1)
        def _():
            acc = None
            for j in range(_F_NB):
                sl = slice(j * _F_DB, (j + 1) * _F_DB)
                hj = jnp.maximum(hh[:, sl] + b1_ref[j][None, :], 0.0)
                pj = jnp.sum(hj * w2_ref[j][None, :], axis=1, keepdims=True)
                acc = pj if acc is None else acc + pj
            lam_ref[...] = jax.nn.sigmoid(acc + b2_ref[...])


def _mega(keys, query, wb1, wb2_row, distances, bb11, W1, b1, W2, b2_11):
    return pl.pallas_call(
        _mega_body,
        grid=(2 * _F_NB,),
        in_specs=[
            pl.BlockSpec((B, K, _F_DB),
                         lambda i: (0, 0, jnp.where(i <= _F_NB,
                                                    jnp.minimum(i, _F_NB - 1),
                                                    _F_NB - 2))),
            pl.BlockSpec((B, _F_DB), lambda i: (0, jnp.minimum(i, _F_NB - 1))),
            pl.BlockSpec((1, _F_DB), lambda i: (0, jnp.minimum(i, _F_NB - 1))),
            pl.BlockSpec((1, _F_DB), lambda i: (0, jnp.minimum(i, _F_NB - 1))),
            pl.BlockSpec((B, K), lambda i: (0, 0)),
            pl.BlockSpec((1, 1), lambda i: (0, 0)),
            pl.BlockSpec((_F_DB, D),
                         lambda i: (jnp.minimum(i, _F_NB - 1), 0)),
            pl.BlockSpec((_F_DB, D),
                         lambda i: (2 * _F_NB - 1
                                    - jnp.clip(i - _F_NB, 0, _F_NB - 1), 0)),
            pl.BlockSpec((_F_NB, _F_DB), lambda i: (0, 0)),
            pl.BlockSpec((_F_NB, _F_DB), lambda i: (0, 0)),
            pl.BlockSpec((1, 1), lambda i: (0, 0)),
        ],
        out_specs=[
            pl.BlockSpec((B, K), lambda i: (0, 0)),
            pl.BlockSpec((B, 1), lambda i: (0, 0)),
        ],
        out_shape=[
            jax.ShapeDtypeStruct((B, K), jnp.float32),
            jax.ShapeDtypeStruct((B, 1), jnp.float32),
        ],
        scratch_shapes=[
            pltpu.VMEM((B, 1), jnp.float32),
            pltpu.VMEM((_F_NB - 2, B, K, _F_DB), jnp.bfloat16),
            pltpu.VMEM((B, D), jnp.float32),
            pltpu.VMEM((B, _F_DB), jnp.float32),
        ],
    )(keys, query, wb1, wb2_row, distances, bb11, W1, W1,
      b1.reshape(_F_NB, _F_DB), W2.reshape(_F_NB, _F_DB), b2_11)


_NC, _NS = 2, 16
_RG = 16
_NVS = 8
_VSEG = V // _NVS
_NRG = B // _RG


@functools.partial(
    pl.kernel,
    mesh=plsc.VectorSubcoreMesh(core_axis_name="c", subcore_axis_name="s"),
    compiler_params=pltpu.CompilerParams(needs_layout_passes=False),
    out_type=jax.ShapeDtypeStruct((B, V), jnp.float32),
    scratch_types=[
        pltpu.VMEM((_RG, K), jnp.int32),
        pltpu.VMEM((_RG, K), jnp.float32),
        pltpu.VMEM((_RG, _VSEG), jnp.float32),
        pltpu.SemaphoreType.DMA,
        pltpu.SemaphoreType.DMA,
    ],
)
def _scatter(vals_hbm, w_hbm, out_hbm, idx_v, w_v, acc_v, sem1, sem2):
    wid = lax.axis_index("s") * _NC + lax.axis_index("c")
    rg = wid % _NRG
    vs = wid // _NRG
    r0 = rg * _RG
    lo = vs * _VSEG

    cp1 = pltpu.async_copy(vals_hbm.at[pl.ds(r0, _RG), :], idx_v, sem1)
    cp2 = pltpu.async_copy(w_hbm.at[pl.ds(r0, _RG), :], w_v, sem2)

    zeros16 = jnp.zeros((16,), jnp.float32)

    @pl.loop(0, _RG)
    def _(r):
        @pl.loop(0, _VSEG, step=16, unroll=8)
        def _(cc):
            acc_v[r, pl.ds(cc, 16)] = zeros16

    cp1.wait()
    cp2.wait()

    lane = lax.iota(jnp.int32, 16)

    @pl.loop(0, K, unroll=4)
    def _(k):
        kk = jnp.full((16,), k, jnp.int32)
        iv = plsc.load_gather(idx_v, [lane, kk])
        wv = plsc.load_gather(w_v, [lane, kk])
        local = iv - lo
        mask = (local >= 0) & (local < _VSEG)
        clamped = jnp.clip(local, 0, _VSEG - 1)
        plsc.addupdate_scatter(acc_v, [lane, clamped], wv, mask=mask)

    pltpu.sync_copy(acc_v, out_hbm.at[pl.ds(r0, _RG), pl.ds(lo, _VSEG)])


def kernel(query, keys, distances, values, Wb, bb, W1, b1, W2, b2):
    wb1 = Wb[:D].reshape(1, D)
    wb2_row = Wb[D:].reshape(1, D)
    bb11 = bb.reshape(1, 1)

    b2_11 = b2.reshape(1, 1)

    w, lam = _mega(keys, query, wb1, wb2_row, distances, bb11, W1,
                   b1, W2, b2_11)

    vals2d = values[..., 0].astype(jnp.int32)
    probs = _scatter(vals2d, w)
    return probs, lam


# speedup vs baseline: 1.2242x; 1.0093x over previous
import functools

import jax
import jax.numpy as jnp
from jax import lax
from jax.experimental import pallas as pl
from jax.experimental.pallas import tpu as pltpu
from jax.experimental.pallas import tpu_sc as plsc

B, K, D, V = 64, 64, 4096, 8192


_F_DB = 256
_F_NB = D // _F_DB


def _mega_body(keys_ref, q_ref, wb1_ref, wb2_ref, dist_ref, bb_ref, w1a_ref,
               w1b_ref, b1_ref, w2_ref, b2_ref,
               w_ref, lam_ref, kd_acc, kcache, hh, wsr, wsp):
    i = pl.program_id(0)

    @pl.when(i < _F_NB)
    def _():
        x = keys_ref[...]

        @pl.when(i < _F_NB - 2)
        def _():
            kcache[jnp.minimum(i, _F_NB - 3)] = x.astype(jnp.bfloat16)

        qp = q_ref[...]
        s1 = jnp.sum(x, axis=1)
        part = (jnp.sum(s1 * wb2_ref[...], axis=1, keepdims=True) * (1.0 / K)
                + jnp.sum(qp * wb1_ref[...], axis=1, keepdims=True))

        @pl.when(i == 0)
        def _():
            kd_acc[...] = part

        @pl.when(i > 0)
        def _():
            kd_acc[...] += part

        pm = jnp.dot(qp.astype(jnp.bfloat16),
                     w1a_ref[...].astype(jnp.bfloat16),
                     preferred_element_type=jnp.float32)

        @pl.when(i == 0)
        def _():
            hh[...] = pm

        @pl.when(i > 0)
        def _():
            hh[...] += pm

    @pl.when(i == _F_NB)
    def _():
        bw = jnp.exp(kd_acc[...] + bb_ref[...])
        sd = -jnp.sqrt(dist_ref[...]) / bw
        m = jnp.max(sd, axis=1, keepdims=True)
        e = jnp.exp(sd - m)
        w = e / jnp.sum(e, axis=1, keepdims=True)
        w_ref[...] = w
        wt = jnp.concatenate([w] * B, axis=1)
        col = lax.broadcasted_iota(jnp.int32, (B, B * K), 1)
        row = lax.broadcasted_iota(jnp.int32, (B, B * K), 0)
        wsp[...] = jnp.where((col // K) == row, wt, 0.0).astype(jnp.bfloat16)

    @pl.when(i >= _F_NB)
    def _():
        r = i - _F_NB

        @pl.when(r <= 1)
        def _():
            kv = keys_ref[...].astype(jnp.bfloat16).reshape(B * K, _F_DB)
            wsr[...] = jnp.dot(wsp[...], kv,
                               preferred_element_type=jnp.float32)

        @pl.when(r > 1)
        def _():
            xk = kcache[jnp.clip(_F_NB - 1 - r, 0, _F_NB - 3)]
            kv = xk.reshape(B * K, _F_DB)
            wsr[...] = jnp.dot(wsp[...], kv,
                               preferred_element_type=jnp.float32)

        hh[...] += jnp.dot(wsr[...].astype(jnp.bfloat16),
                           w1b_ref[...].astype(jnp.bfloat16),
                           preferred_element_type=jnp.float32)

        @pl.when(r == _F_NB - 1)
        def _():
            acc = None
            for j in range(_F_NB):
                sl = slice(j * _F_DB, (j + 1) * _F_DB)
                hj = jnp.maximum(hh[:, sl] + b1_ref[j][None, :], 0.0)
                pj = jnp.sum(hj * w2_ref[j][None, :], axis=1, keepdims=True)
                acc = pj if acc is None else acc + pj
            lam_ref[...] = jax.nn.sigmoid(acc + b2_ref[...])


def _mega(keys, query, wb1, wb2_row, distances, bb11, W1, b1, W2, b2_11):
    return pl.pallas_call(
        _mega_body,
        grid=(2 * _F_NB,),
        in_specs=[
            pl.BlockSpec((B, K, _F_DB),
                         lambda i: (0, 0, jnp.where(i <= _F_NB,
                                                    jnp.minimum(i, _F_NB - 1),
                                                    _F_NB - 2))),
            pl.BlockSpec((B, _F_DB), lambda i: (0, jnp.minimum(i, _F_NB - 1))),
            pl.BlockSpec((1, _F_DB), lambda i: (0, jnp.minimum(i, _F_NB - 1))),
            pl.BlockSpec((1, _F_DB), lambda i: (0, jnp.minimum(i, _F_NB - 1))),
            pl.BlockSpec((B, K), lambda i: (0, 0)),
            pl.BlockSpec((1, 1), lambda i: (0, 0)),
            pl.BlockSpec((_F_DB, D),
                         lambda i: (jnp.minimum(i, _F_NB - 1), 0)),
            pl.BlockSpec((_F_DB, D),
                         lambda i: (2 * _F_NB - 1
                                    - jnp.clip(i - _F_NB, 0, _F_NB - 1), 0)),
            pl.BlockSpec((_F_NB, _F_DB), lambda i: (0, 0)),
            pl.BlockSpec((_F_NB, _F_DB), lambda i: (0, 0)),
            pl.BlockSpec((1, 1), lambda i: (0, 0)),
        ],
        out_specs=[
            pl.BlockSpec((B, K), lambda i: (0, 0)),
            pl.BlockSpec((B, 1), lambda i: (0, 0)),
        ],
        out_shape=[
            jax.ShapeDtypeStruct((B, K), jnp.float32),
            jax.ShapeDtypeStruct((B, 1), jnp.float32),
        ],
        scratch_shapes=[
            pltpu.VMEM((B, 1), jnp.float32),
            pltpu.VMEM((_F_NB - 2, B, K, _F_DB), jnp.bfloat16),
            pltpu.VMEM((B, D), jnp.float32),
            pltpu.VMEM((B, _F_DB), jnp.float32),
            pltpu.VMEM((B, B * K), jnp.bfloat16),
        ],
    )(keys, query, wb1, wb2_row, distances, bb11, W1, W1,
      b1.reshape(_F_NB, _F_DB), W2.reshape(_F_NB, _F_DB), b2_11)


_NC, _NS = 2, 16
_RG = 16
_NVS = 8
_VSEG = V // _NVS
_NRG = B // _RG


@functools.partial(
    pl.kernel,
    mesh=plsc.VectorSubcoreMesh(core_axis_name="c", subcore_axis_name="s"),
    compiler_params=pltpu.CompilerParams(needs_layout_passes=False),
    out_type=jax.ShapeDtypeStruct((B, V), jnp.float32),
    scratch_types=[
        pltpu.VMEM((_RG, K), jnp.int32),
        pltpu.VMEM((_RG, K), jnp.float32),
        pltpu.VMEM((_RG, _VSEG), jnp.float32),
        pltpu.SemaphoreType.DMA,
        pltpu.SemaphoreType.DMA,
    ],
)
def _scatter(vals_hbm, w_hbm, out_hbm, idx_v, w_v, acc_v, sem1, sem2):
    wid = lax.axis_index("s") * _NC + lax.axis_index("c")
    rg = wid % _NRG
    vs = wid // _NRG
    r0 = rg * _RG
    lo = vs * _VSEG

    cp1 = pltpu.async_copy(vals_hbm.at[pl.ds(r0, _RG), :], idx_v, sem1)
    cp2 = pltpu.async_copy(w_hbm.at[pl.ds(r0, _RG), :], w_v, sem2)

    zeros16 = jnp.zeros((16,), jnp.float32)

    @pl.loop(0, _RG)
    def _(r):
        @pl.loop(0, _VSEG, step=16, unroll=8)
        def _(cc):
            acc_v[r, pl.ds(cc, 16)] = zeros16

    cp1.wait()
    cp2.wait()

    lane = lax.iota(jnp.int32, 16)

    @pl.loop(0, K, unroll=4)
    def _(k):
        kk = jnp.full((16,), k, jnp.int32)
        iv = plsc.load_gather(idx_v, [lane, kk])
        wv = plsc.load_gather(w_v, [lane, kk])
        local = iv - lo
        mask = (local >= 0) & (local < _VSEG)
        clamped = jnp.clip(local, 0, _VSEG - 1)
        plsc.addupdate_scatter(acc_v, [lane, clamped], wv, mask=mask)

    pltpu.sync_copy(acc_v, out_hbm.at[pl.ds(r0, _RG), pl.ds(lo, _VSEG)])


def kernel(query, keys, distances, values, Wb, bb, W1, b1, W2, b2):
    wb1 = Wb[:D].reshape(1, D)
    wb2_row = Wb[D:].reshape(1, D)
    bb11 = bb.reshape(1, 1)

    b2_11 = b2.reshape(1, 1)

    w, lam = _mega(keys, query, wb1, wb2_row, distances, bb11, W1,
                   b1, W2, b2_11)

    vals2d = values[..., 0].astype(jnp.int32)
    probs = _scatter(vals2d, w)
    return probs, lam


# speedup vs baseline: 1.2384x; 1.0116x over previous
import functools

import jax
import jax.numpy as jnp
from jax import lax
from jax.experimental import pallas as pl
from jax.experimental.pallas import tpu as pltpu
from jax.experimental.pallas import tpu_sc as plsc

B, K, D, V = 64, 64, 4096, 8192


_F_DB = 256
_F_NB = D // _F_DB


def _fused_body(keys_ref, q_ref, wb1_ref, wb2_ref, dist_ref, bb_ref, w1a_ref,
                hq_ref, wsum_ref, w_ref, kd_acc, kcache):
    i = pl.program_id(0)

    @pl.when(i < _F_NB)
    def _():
        x = keys_ref[...]
        kcache[i] = x.astype(jnp.bfloat16)
        part = jnp.sum(jnp.sum(x * wb2_ref[...][None, :, :], axis=1),
                       axis=1, keepdims=True)

        @pl.when(i == 0)
        def _():
            kd_acc[...] = part

        @pl.when(i > 0)
        def _():
            kd_acc[...] += part

        hq_ref[...] = jnp.dot(q_ref[...].astype(jnp.bfloat16),
                              w1a_ref[...].astype(jnp.bfloat16),
                              preferred_element_type=jnp.float32)

    @pl.when(i == _F_NB)
    def _():
        qd = jnp.dot(q_ref[...], wb1_ref[...],
                     preferred_element_type=jnp.float32)
        t2 = kd_acc[...] * (1.0 / K)
        bw = jnp.exp(qd + t2 + bb_ref[...])
        sd = -jnp.sqrt(dist_ref[...]) / bw
        m = jnp.max(sd, axis=1, keepdims=True)
        e = jnp.exp(sd - m)
        w = e / jnp.sum(e, axis=1, keepdims=True)
        w_ref[...] = w
        for j in range(_F_NB):
            xk = kcache[j].astype(jnp.float32)
            wsum_ref[:, j * _F_DB:(j + 1) * _F_DB] = jnp.sum(
                w[:, :, None] * xk, axis=1)


def _fused(keys, query, wb1, wb2_row, distances, bb11, W1):
    return pl.pallas_call(
        _fused_body,
        grid=(_F_NB + 1,),
        in_specs=[
            pl.BlockSpec((B, K, _F_DB),
                         lambda i: (0, 0, jnp.minimum(i, _F_NB - 1))),
            pl.BlockSpec((B, D), lambda i: (0, 0)),
            pl.BlockSpec((D, 1), lambda i: (0, 0)),
            pl.BlockSpec((1, _F_DB), lambda i: (0, jnp.minimum(i, _F_NB - 1))),
            pl.BlockSpec((B, K), lambda i: (0, 0)),
            pl.BlockSpec((1, 1), lambda i: (0, 0)),
            pl.BlockSpec((D, _F_DB),
                         lambda i: (0, jnp.minimum(i, _F_NB - 1))),
        ],
        out_specs=[
            pl.BlockSpec((B, _F_DB),
                         lambda i: (0, jnp.minimum(i, _F_NB - 1))),
            pl.BlockSpec((B, D), lambda i: (0, 0)),
            pl.BlockSpec((B, K), lambda i: (0, 0)),
        ],
        out_shape=[
            jax.ShapeDtypeStruct((B, D), jnp.float32),
            jax.ShapeDtypeStruct((B, D), jnp.float32),
            jax.ShapeDtypeStruct((B, K), jnp.float32),
        ],
        scratch_shapes=[
            pltpu.VMEM((B, 1), jnp.float32),
            pltpu.VMEM((_F_NB, B, K, _F_DB), jnp.bfloat16),
        ],
    )(keys, query, wb1, wb2_row, distances, bb11, W1)


_NC, _NS = 2, 16
_RG = 16
_NVS = 8
_VSEG = V // _NVS
_NRG = B // _RG


@functools.partial(
    pl.kernel,
    mesh=plsc.VectorSubcoreMesh(core_axis_name="c", subcore_axis_name="s"),
    compiler_params=pltpu.CompilerParams(needs_layout_passes=False),
    out_type=jax.ShapeDtypeStruct((B, V), jnp.float32),
    scratch_types=[
        pltpu.VMEM((_RG, K), jnp.int32),
        pltpu.VMEM((_RG, K), jnp.float32),
        pltpu.VMEM((_RG, _VSEG), jnp.float32),
    ],
)
def _scatter(vals_hbm, w_hbm, out_hbm, idx_v, w_v, acc_v):
    wid = lax.axis_index("s") * _NC + lax.axis_index("c")
    rg = wid % _NRG
    vs = wid // _NRG
    r0 = rg * _RG
    lo = vs * _VSEG

    pltpu.sync_copy(vals_hbm.at[pl.ds(r0, _RG), :], idx_v)
    pltpu.sync_copy(w_hbm.at[pl.ds(r0, _RG), :], w_v)

    @pl.loop(0, _RG)
    def _(r):
        @pl.loop(0, _VSEG, step=16)
        def _(cc):
            acc_v[r, pl.ds(cc, 16)] = jnp.zeros((16,), jnp.float32)

    lane = lax.iota(jnp.int32, 16)

    @pl.loop(0, K)
    def _(k):
        kk = jnp.full((16,), k, jnp.int32)
        iv = plsc.load_gather(idx_v, [lane, kk])
        wv = plsc.load_gather(w_v, [lane, kk])
        local = iv - lo
        mask = (local >= 0) & (local < _VSEG)
        clamped = jnp.clip(local, 0, _VSEG - 1)
        plsc.addupdate_scatter(acc_v, [lane, clamped], wv, mask=mask)

    pltpu.sync_copy(acc_v, out_hbm.at[pl.ds(r0, _RG), pl.ds(lo, _VSEG)])


_MLP_NB = 512


def _mlp_body(hq_ref, ws_ref, w1b_ref, b1_ref, w2_ref, b2_ref, out_ref):
    j = pl.program_id(0)
    h = hq_ref[...] + jnp.dot(ws_ref[...].astype(jnp.bfloat16),
                              w1b_ref[...].astype(jnp.bfloat16),
                              preferred_element_type=jnp.float32)
    h = jnp.maximum(h + b1_ref[...], 0.0)
    part = jnp.dot(h, w2_ref[...], preferred_element_type=jnp.float32)

    @pl.when(j == 0)
    def _():
        out_ref[...] = part

    @pl.when(j > 0)
    def _():
        out_ref[...] += part

    @pl.when(j == D // _MLP_NB - 1)
    def _():
        out_ref[...] = jax.nn.sigmoid(out_ref[...] + b2_ref[...])


def _mlp(hq, wsum, W1, b1_row, W2, b2_11):
    return pl.pallas_call(
        _mlp_body,
        grid=(D // _MLP_NB,),
        in_specs=[
            pl.BlockSpec((B, _MLP_NB), lambda j: (0, j)),
            pl.BlockSpec((B, D), lambda j: (0, 0)),
            pl.BlockSpec((D, _MLP_NB), lambda j: (1, j)),
            pl.BlockSpec((1, _MLP_NB), lambda j: (0, j)),
            pl.BlockSpec((_MLP_NB, 1), lambda j: (j, 0)),
            pl.BlockSpec((1, 1), lambda j: (0, 0)),
        ],
        out_specs=pl.BlockSpec((B, 1), lambda j: (0, 0)),
        out_shape=jax.ShapeDtypeStruct((B, 1), jnp.float32),
    )(hq, wsum, W1, b1_row, W2, b2_11)


def kernel(query, keys, distances, values, Wb, bb, W1, b1, W2, b2):
    wb1 = Wb[:D]
    wb2_row = Wb[D:].reshape(1, D)
    bb11 = bb.reshape(1, 1)
    b1_row = b1.reshape(1, D)
    b2_11 = b2.reshape(1, 1)

    hq, wsum, w = _fused(keys, query, wb1, wb2_row, distances, bb11, W1)

    vals2d = values[..., 0].astype(jnp.int32)
    probs = _scatter(vals2d, w)

    lam = _mlp(hq, wsum, W1, b1_row, W2, b2_11)
    return probs, lam
